# Initial kernel scaffold; baseline (speedup 1.0000x reference)
import jax
import jax.numpy as jnp
from jax.experimental import pallas as pl


def kernel(input_ids, emb_table, W, b):
    raise NotImplementedError("write your pallas kernel here")


# speedup vs baseline: 2.9701x; 2.9701x over previous
import functools

import jax
import jax.numpy as jnp
from jax import lax
from jax.experimental import pallas as pl
from jax.experimental.pallas import tpu as pltpu
from jax.experimental.pallas import tpu_sc as plsc

_B = 16384
_L = 200
_D = 64
_NC = 2
_NS = 16
_NW = _NC * _NS
_RPT = _B // _NW
_SPLIT = 128
_REST = _L - _SPLIT


def _sigmoid(x):
    return 1.0 / (1.0 + jnp.exp(-x))


def _body(idx_hbm, wb_hbm, table_hbm, out_hbm, idx_v, buf0, buf1, wb_v,
          out_v, out_smem, sem0, sem1):
    wid = lax.axis_index("s") * _NC + lax.axis_index("c")
    base = wid * _RPT

    pltpu.sync_copy(wb_hbm, wb_v)
    pltpu.sync_copy(idx_hbm.at[pl.ds(base, _RPT)], idx_v)

    w0 = wb_v[pl.ds(0, 16)]
    w1 = wb_v[pl.ds(16, 16)]
    w2 = wb_v[pl.ds(32, 16)]
    w3 = wb_v[pl.ds(48, 16)]
    bvec = wb_v[pl.ds(64, 16)]

    def fire(r, buf, sem):
        pltpu.async_copy(
            table_hbm.at[idx_v.at[r, pl.ds(0, _SPLIT)]],
            buf.at[pl.ds(0, _SPLIT)], sem)
        pltpu.async_copy(
            table_hbm.at[idx_v.at[r, pl.ds(_SPLIT, _REST)]],
            buf.at[pl.ds(_SPLIT, _REST)], sem)

    def wait(buf, sem):
        pltpu.make_async_copy(table_hbm.at[pl.ds(0, _L)], buf, sem).wait()

    def process(r, buf):
        def red(j, accs):
            a0, a1, a2, a3 = accs
            row = buf.at[j]
            return (a0 + row[pl.ds(0, 16)],
                    a1 + row[pl.ds(16, 16)],
                    a2 + row[pl.ds(32, 16)],
                    a3 + row[pl.ds(48, 16)])

        z = jnp.zeros((16,), jnp.float32)
        a0, a1, a2, a3 = lax.fori_loop(0, _L, red, (z, z, z, z), unroll=8)
        inv = jnp.float32(1.0 / _L)
        h0 = _sigmoid(a0 * inv)
        h1 = _sigmoid(a1 * inv)
        h2 = _sigmoid(a2 * inv)
        h3 = _sigmoid(a3 * inv)
        t = h0 * w0 + h1 * w1 + h2 * w2 + h3 * w3 + bvec
        out_smem[r] = jnp.sum(t)

    fire(0, buf0, sem0)

    def loop(i, carry):
        r0 = 2 * i
        fire(r0 + 1, buf1, sem1)
        wait(buf0, sem0)
        process(r0, buf0)

        @pl.when(r0 + 2 < _RPT)
        def _():
            fire(r0 + 2, buf0, sem0)

        wait(buf1, sem1)
        process(r0 + 1, buf1)
        return carry

    lax.fori_loop(0, _RPT // 2, loop, 0)

    lane = lax.iota(jnp.int32, 16)

    def pack(g, carry):
        def ins(k, v):
            return jnp.where(lane == k, out_smem[g * 16 + k], v)

        v = lax.fori_loop(0, 16, ins, jnp.zeros((16,), jnp.float32))
        out_v[pl.ds(g * 16, 16)] = _sigmoid(v)
        return carry

    lax.fori_loop(0, _RPT // 16, pack, 0)

    pltpu.sync_copy(out_v, out_hbm.at[pl.ds(base, _RPT)])


@jax.jit
def _run(idx, wb, table):
    mesh = plsc.VectorSubcoreMesh(core_axis_name="c", subcore_axis_name="s")
    f = pl.kernel(
        _body,
        out_type=jax.ShapeDtypeStruct((_B,), jnp.float32),
        mesh=mesh,
        compiler_params=pltpu.CompilerParams(
            needs_layout_passes=False, use_tc_tiling_on_sc=False),
        scratch_types=[
            pltpu.VMEM((_RPT, _L), jnp.int32),
            pltpu.VMEM((_L, _D), jnp.float32),
            pltpu.VMEM((_L, _D), jnp.float32),
            pltpu.VMEM((80,), jnp.float32),
            pltpu.VMEM((_RPT,), jnp.float32),
            pltpu.SMEM((_RPT,), jnp.float32),
            pltpu.SemaphoreType.DMA,
            pltpu.SemaphoreType.DMA,
        ],
    )
    return f(idx, wb, table)


def kernel(input_ids, emb_table, W, b):
    wb = jnp.concatenate(
        [W.reshape(-1), b.reshape(-1),
         jnp.zeros((15,), jnp.float32)]).astype(jnp.float32)
    out = _run(input_ids, wb, emb_table)
    return out.reshape(_B, 1)
